# baseline (speedup 1.0000x reference)
import functools

import jax
import jax.numpy as jnp
from jax import lax
from jax.experimental import pallas as pl
from jax.experimental.pallas import tpu as pltpu
from jax.experimental.pallas import tpu_sc as plsc

_N_TABLES = 26
_VOCAB = 100000
_D = 32
_NC = 2
_NS = 16
_NW = _NC * _NS
_CB = 64


def _body(idx_hbm, tables_hbm, out_hbm, idx_a, idx_b, rows_a, rows_b, gsem):
    n_rows = idx_hbm.shape[0]
    bpw = n_rows // _NW
    n = _CB * _N_TABLES
    n_chunks = bpw // n
    wid = lax.axis_index("s") * _NC + lax.axis_index("c")
    base = wid * bpw
    bufs = (rows_a, rows_b)
    idxs = (idx_a, idx_b)

    pltpu.sync_copy(idx_hbm.at[pl.ds(base, n)], idxs[0])
    cp = pltpu.async_copy(tables_hbm.at[idxs[0]], bufs[0], gsem)
    for c in range(n_chunks):
        if c + 1 < n_chunks:
            pltpu.sync_copy(
                idx_hbm.at[pl.ds(base + (c + 1) * n, n)], idxs[(c + 1) % 2])
            nxt = pltpu.async_copy(tables_hbm.at[idxs[(c + 1) % 2]],
                                   bufs[(c + 1) % 2], gsem)
        cp.wait()
        pltpu.sync_copy(bufs[c % 2], out_hbm.at[pl.ds(base + c * n, n)])
        if c + 1 < n_chunks:
            cp = nxt


def kernel(x, tables):
    b = x.shape[0]
    n = _CB * _N_TABLES
    idx = (x.astype(jnp.int32)[:, None]
           + (jnp.arange(_N_TABLES, dtype=jnp.int32) * _VOCAB)[None, :]
           ).reshape(-1)
    run = pl.kernel(
        _body,
        out_type=jax.ShapeDtypeStruct((b * _N_TABLES, _D), jnp.float32),
        mesh=plsc.VectorSubcoreMesh(
            core_axis_name="c", subcore_axis_name="s",
            num_cores=_NC, num_subcores=_NS),
        scratch_types=[
            pltpu.VMEM((n,), jnp.int32),
            pltpu.VMEM((n,), jnp.int32),
            pltpu.VMEM((n, _D), jnp.float32),
            pltpu.VMEM((n, _D), jnp.float32),
            pltpu.SemaphoreType.DMA,
        ],
        compiler_params=pltpu.CompilerParams(use_tc_tiling_on_sc=False, needs_layout_passes=False),
    )
    out = run(idx, tables.reshape(_N_TABLES * _VOCAB, _D))
    return out.reshape(b, _N_TABLES, _D)


# speedup vs baseline: 4.2861x; 4.2861x over previous
import functools

import jax
import jax.numpy as jnp
from jax import lax
from jax.experimental import pallas as pl
from jax.experimental.pallas import tpu as pltpu
from jax.experimental.pallas import tpu_sc as plsc

_N_TABLES = 26
_VOCAB = 100000
_D = 32
_B = 16384
_NC = 2
_NS = 16
_NW = _NC * _NS
_BH = _B // 2


def _body(x_hbm, tab_hbm, out_hbm, row_v, x_v, res_v, sem):
    wid = lax.axis_index("s") * _NC + lax.axis_index("c")
    dt = wid // 8
    r = wid % 8

    for t in range(_N_TABLES):
        pltpu.sync_copy(tab_hbm.at[t, wid], row_v)
        for h in range(2):
            pltpu.sync_copy(x_hbm.at[pl.ds(h * _BH, _BH)], x_v)

            def sel(row, carry):
                for c in range(8):
                    xv = x_v[pl.ds(row * 128 + c * 16, 16)]
                    vals = plsc.load_gather(row_v, [xv])
                    res_v[row, pl.ds(c * 16, 16)] = vals
                return carry

            lax.fori_loop(0, _BH // 128, sel, 0)
            pltpu.sync_copy(
                res_v,
                out_hbm.at[t, dt, pl.ds(h * (_BH // 128), _BH // 128), r])


def kernel(x, tables):
    tab_t = jnp.transpose(tables, (0, 2, 1))
    run = pl.kernel(
        _body,
        out_type=jax.ShapeDtypeStruct((_N_TABLES, 4, _B // 128, 8, 128),
                                      jnp.float32),
        mesh=plsc.VectorSubcoreMesh(
            core_axis_name="c", subcore_axis_name="s",
            num_cores=_NC, num_subcores=_NS),
        scratch_types=[
            pltpu.VMEM((_VOCAB,), jnp.float32),
            pltpu.VMEM((_BH,), jnp.int32),
            pltpu.VMEM((_BH // 128, 128), jnp.float32),
            pltpu.SemaphoreType.DMA,
        ],
        compiler_params=pltpu.CompilerParams(
            use_tc_tiling_on_sc=True, needs_layout_passes=False),
    )
    out5d = run(x.astype(jnp.int32), tab_t)
    out = out5d.transpose(2, 4, 0, 1, 3).reshape(_B, _N_TABLES, _D)
    return out


# speedup vs baseline: 5.4084x; 1.2618x over previous
import functools

import jax
import jax.numpy as jnp
from jax import lax
from jax.experimental import pallas as pl
from jax.experimental.pallas import tpu as pltpu
from jax.experimental.pallas import tpu_sc as plsc

_N_TABLES = 26
_VOCAB = 100000
_D = 32
_B = 16384
_NC = 2
_NS = 16
_NW = _NC * _NS
_Q = _B // 4
_QR = _Q // 128


def _body(x_hbm, tab_hbm, out_hbm, row_v, x_v, res_a, res_b, osem):
    wid = lax.axis_index("s") * _NC + lax.axis_index("c")
    dt = wid // 8
    r = wid % 8
    res = (res_a, res_b)
    pending = [None, None]

    pltpu.sync_copy(x_hbm, x_v)
    step = 0
    for t in range(_N_TABLES):
        pltpu.sync_copy(tab_hbm.at[t, wid], row_v)
        for h in range(4):
            slot = step % 2
            buf = res[slot]
            if pending[slot] is not None:
                pending[slot].wait()

            def sel(row, carry):
                for c in range(8):
                    xv = x_v[pl.ds(h * _Q + row * 128 + c * 16, 16)]
                    buf[row, pl.ds(c * 16, 16)] = plsc.load_gather(
                        row_v, [xv])
                return carry

            lax.fori_loop(0, _QR, sel, 0)

            pending[slot] = pltpu.async_copy(
                buf, out_hbm.at[t, dt, pl.ds(h * _QR, _QR), r], osem)
            step += 1
    for cp in pending:
        if cp is not None:
            cp.wait()


def kernel(x, tables):
    tab_t = jnp.transpose(tables, (0, 2, 1))
    run = pl.kernel(
        _body,
        out_type=jax.ShapeDtypeStruct((_N_TABLES, 4, _B // 128, 8, 128),
                                      jnp.float32),
        mesh=plsc.VectorSubcoreMesh(
            core_axis_name="c", subcore_axis_name="s",
            num_cores=_NC, num_subcores=_NS),
        scratch_types=[
            pltpu.VMEM((_VOCAB,), jnp.float32),
            pltpu.VMEM((_B,), jnp.int32),
            pltpu.VMEM((_QR, 128), jnp.float32),
            pltpu.VMEM((_QR, 128), jnp.float32),
            pltpu.SemaphoreType.DMA,
        ],
        compiler_params=pltpu.CompilerParams(
            use_tc_tiling_on_sc=True, needs_layout_passes=False),
    )
    out5d = run(x.astype(jnp.int32), tab_t)
    out = out5d.transpose(2, 4, 0, 1, 3).reshape(_B, _N_TABLES, _D)
    return out
